# baseline (speedup 1.0000x reference)
import jax
import jax.numpy as jnp
import numpy as np
from jax.experimental import pallas as pl
from jax.experimental.pallas import tpu as pltpu

_F = 26
_D = 16
_PAIR_LIST = [(i, j) for i in range(_F - 1) for j in range(i + 1, _F)]
_P = len(_PAIR_LIST)

_ONEHOT_I = np.zeros((_F, _P), np.float32)
_ONEHOT_J = np.zeros((_F, _P), np.float32)
for _p, (_i, _j) in enumerate(_PAIR_LIST):
    _ONEHOT_I[_i, _p] = 1.0
    _ONEHOT_J[_j, _p] = 1.0

_BB = 32


def _body(x_ref, oi_ref, oj_ref, out_ref):
    bb = x_ref.shape[0]
    xt = jnp.transpose(x_ref[...], (0, 2, 1)).reshape(bb * _D, _F)
    u = jnp.dot(xt, oi_ref[...], preferred_element_type=jnp.float32)
    v = jnp.dot(xt, oj_ref[...], preferred_element_type=jnp.float32)
    u4 = u.reshape(bb, _D, 1, _P)
    v4 = v.reshape(bb, 1, _D, _P)
    out_ref[...] = u4 * v4


def kernel(inputs):
    B, F, D = inputs.shape
    grid = (B // _BB,)
    return pl.pallas_call(
        _body,
        grid=grid,
        in_specs=[
            pl.BlockSpec((_BB, F, D), lambda i: (i, 0, 0)),
            pl.BlockSpec((F, _P), lambda i: (0, 0)),
            pl.BlockSpec((F, _P), lambda i: (0, 0)),
        ],
        out_specs=pl.BlockSpec((_BB, D, D, _P), lambda i: (i, 0, 0, 0)),
        out_shape=jax.ShapeDtypeStruct((B, D, D, _P), jnp.float32),
        compiler_params=pltpu.CompilerParams(
            dimension_semantics=("parallel",),
        ),
    )(inputs, jnp.asarray(_ONEHOT_I), jnp.asarray(_ONEHOT_J))
